# baseline (speedup 1.0000x reference)
import math

import jax
import jax.numpy as jnp
import numpy as np
from jax import lax
from jax.experimental import pallas as pl
from jax.experimental.pallas import tpu as pltpu
from jax.experimental.pallas import tpu_sc as plsc

VOCAB_SIZE = 1000000
DIM = 64
SEQ_LEN = 200
BATCH = 1024

NUM_WORKERS = 32
BBLK = BATCH // NUM_WORKERS
NBUF = 8
D_PE = 3
D_WR = 5


def _position_embedding_np():
    i = np.arange(SEQ_LEN, dtype=np.float64)[:, None]
    j = np.arange(DIM, dtype=np.float64)[None, :]
    even_mask = (np.arange(DIM) % 2 == 0)[None, :]
    temp_even = np.exp(-(j / DIM) * math.log(10000.0))
    temp_odd = np.exp(-((j - 1.0) / DIM) * math.log(10000.0))
    pe = np.where(even_mask, np.sin(i * temp_even), np.cos(i * temp_odd))
    return pe.astype(np.float32)


_PE = _position_embedding_np()
_POS = np.repeat(np.arange(SEQ_LEN, dtype=np.int32)[:, None], BBLK, axis=1)


def _sc_body(xt_hbm, pos_hbm, table_hbm, pe_hbm, out_hbm,
             idx_v, pos_v, rows_v, gsem, psem, wsem):
    wid = lax.axis_index("s") * 2 + lax.axis_index("c")
    b0 = wid * BBLK

    pltpu.sync_copy(xt_hbm.at[:, pl.ds(b0, BBLK)], idx_v)
    pltpu.sync_copy(pos_hbm, pos_v)

    def slot(g):
        return lax.rem(g, NBUF)

    def fire_tbl(l):
        pltpu.async_copy(table_hbm.at[idx_v.at[l]], rows_v.at[slot(l)],
                         gsem.at[slot(l)])

    def wait_tbl(l):
        pltpu.make_async_copy(table_hbm.at[idx_v.at[l]], rows_v.at[slot(l)],
                              gsem.at[slot(l)]).wait()

    def fire_pe(l):
        pltpu.async_copy(pe_hbm.at[pos_v.at[l]], rows_v.at[slot(l)],
                         psem.at[slot(l)], add=True)

    def wait_pe(l):
        pltpu.make_async_copy(pe_hbm.at[pos_v.at[l]], rows_v.at[slot(l)],
                              psem.at[slot(l)]).wait()

    def fire_write(l):
        pltpu.async_copy(rows_v.at[slot(l)],
                         out_hbm.at[pl.ds(b0, BBLK), l], wsem.at[slot(l)])

    def wait_write(l):
        pltpu.make_async_copy(rows_v.at[slot(l)],
                              out_hbm.at[pl.ds(b0, BBLK), l],
                              wsem.at[slot(l)]).wait()

    for i in range(NBUF):
        fire_tbl(i)
        if i >= D_PE:
            wait_tbl(i - D_PE)
            fire_pe(i - D_PE)
        if i >= D_WR:
            wait_pe(i - D_WR)
            fire_write(i - D_WR)

    def steady(i, carry):
        wait_write(i - NBUF)
        fire_tbl(i)
        wait_tbl(i - D_PE)
        fire_pe(i - D_PE)
        wait_pe(i - D_WR)
        fire_write(i - D_WR)
        return carry

    lax.fori_loop(NBUF, SEQ_LEN, steady, 0)

    for l in range(SEQ_LEN - D_PE, SEQ_LEN):
        wait_tbl(l)
        fire_pe(l)
    for l in range(SEQ_LEN - D_WR, SEQ_LEN):
        wait_pe(l)
        fire_write(l)
    for l in range(SEQ_LEN - NBUF, SEQ_LEN):
        wait_write(l)


@jax.jit
def kernel(x, table):
    xt = jnp.swapaxes(x, 0, 1)
    pe = jnp.asarray(_PE)
    pos = jnp.asarray(_POS)
    run = pl.kernel(
        _sc_body,
        out_type=jax.ShapeDtypeStruct((BATCH, SEQ_LEN, DIM), jnp.float32),
        mesh=plsc.VectorSubcoreMesh(core_axis_name="c", subcore_axis_name="s"),
        scratch_types=[
            pltpu.VMEM((SEQ_LEN, BBLK), jnp.int32),
            pltpu.VMEM((SEQ_LEN, BBLK), jnp.int32),
            pltpu.VMEM((NBUF, BBLK, DIM), jnp.float32),
            pltpu.SemaphoreType.DMA((NBUF,)),
            pltpu.SemaphoreType.DMA((NBUF,)),
            pltpu.SemaphoreType.DMA((NBUF,)),
        ],
        compiler_params=pltpu.CompilerParams(use_tc_tiling_on_sc=False),
    )
    return run(xt, pos, table, pe)


# speedup vs baseline: 1.1148x; 1.1148x over previous
import math

import jax
import jax.numpy as jnp
import numpy as np
from jax import lax
from jax.experimental import pallas as pl
from jax.experimental.pallas import tpu as pltpu
from jax.experimental.pallas import tpu_sc as plsc

VOCAB_SIZE = 1000000
DIM = 64
SEQ_LEN = 200
BATCH = 1024

NUM_WORKERS = 32
BBLK = BATCH // NUM_WORKERS
LCHUNK = 8
N_CHUNKS = SEQ_LEN // LCHUNK
NBUF = 6
D_PE = 2
D_WR = 4


def _position_embedding_np():
    i = np.arange(SEQ_LEN, dtype=np.float64)[:, None]
    j = np.arange(DIM, dtype=np.float64)[None, :]
    even_mask = (np.arange(DIM) % 2 == 0)[None, :]
    temp_even = np.exp(-(j / DIM) * math.log(10000.0))
    temp_odd = np.exp(-((j - 1.0) / DIM) * math.log(10000.0))
    pe = np.where(even_mask, np.sin(i * temp_even), np.cos(i * temp_odd))
    return pe.astype(np.float32)


_PE = _position_embedding_np()
_POS = (np.arange(SEQ_LEN, dtype=np.int32)[:, None]
        .repeat(BBLK, axis=1).reshape(N_CHUNKS, 2, 128))


def _sc_body(xp_hbm, pos_hbm, table_hbm, pe_hbm, out_hbm,
             idx_v, pos_v, rows_v, gsem, psem, wsem):
    wid = lax.axis_index("s") * 2 + lax.axis_index("c")
    bt = wid // 4
    lb0 = (wid % 4) * BBLK
    b0 = bt * 128 + lb0

    for s in range(LCHUNK):
        pltpu.sync_copy(xp_hbm.at[:, bt, s, pl.ds(lb0, BBLK)],
                        idx_v.at[:, s // 4, pl.ds((s % 4) * BBLK, BBLK)])
    pltpu.sync_copy(pos_hbm, pos_v)

    def slot(g):
        return lax.rem(g, NBUF)

    def fire_tbl(c):
        for h in range(2):
            pltpu.async_copy(table_hbm.at[idx_v.at[c, h]],
                             rows_v.at[slot(c), h], gsem.at[slot(c)])

    def wait_tbl(c):
        for h in range(2):
            pltpu.make_async_copy(table_hbm.at[idx_v.at[c, h]],
                                  rows_v.at[slot(c), h],
                                  gsem.at[slot(c)]).wait()

    def fire_pe(c):
        for h in range(2):
            pltpu.async_copy(pe_hbm.at[pos_v.at[c, h]],
                             rows_v.at[slot(c), h], psem.at[slot(c)],
                             add=True)

    def wait_pe(c):
        for h in range(2):
            pltpu.make_async_copy(pe_hbm.at[pos_v.at[c, h]],
                                  rows_v.at[slot(c), h],
                                  psem.at[slot(c)]).wait()

    def fire_write(c):
        for s in range(LCHUNK):
            pltpu.async_copy(
                rows_v.at[slot(c), s // 4, pl.ds((s % 4) * BBLK, BBLK)],
                out_hbm.at[c * LCHUNK + s, pl.ds(b0, BBLK)],
                wsem.at[slot(c)])

    def wait_write(c):
        for s in range(LCHUNK):
            pltpu.make_async_copy(
                rows_v.at[slot(c), s // 4, pl.ds((s % 4) * BBLK, BBLK)],
                out_hbm.at[c * LCHUNK + s, pl.ds(b0, BBLK)],
                wsem.at[slot(c)]).wait()

    for i in range(NBUF):
        fire_tbl(i)
        if i >= D_PE:
            wait_tbl(i - D_PE)
            fire_pe(i - D_PE)
        if i >= D_WR:
            wait_pe(i - D_WR)
            fire_write(i - D_WR)

    def steady(i, carry):
        wait_write(i - NBUF)
        fire_tbl(i)
        wait_tbl(i - D_PE)
        fire_pe(i - D_PE)
        wait_pe(i - D_WR)
        fire_write(i - D_WR)
        return carry

    lax.fori_loop(NBUF, N_CHUNKS, steady, 0)

    for c in range(N_CHUNKS - D_PE, N_CHUNKS):
        wait_tbl(c)
        fire_pe(c)
    for c in range(N_CHUNKS - D_WR, N_CHUNKS):
        wait_pe(c)
        fire_write(c)
    for c in range(N_CHUNKS - NBUF, N_CHUNKS):
        wait_write(c)


@jax.jit
def kernel(x, table):
    xp = x.T.reshape(N_CHUNKS, LCHUNK, BATCH // 128, 128).transpose(0, 2, 1, 3)
    pe = jnp.asarray(_PE)
    pos = jnp.asarray(_POS)
    run = pl.kernel(
        _sc_body,
        out_type=jax.ShapeDtypeStruct((SEQ_LEN, BATCH, DIM), jnp.float32),
        mesh=plsc.VectorSubcoreMesh(core_axis_name="c", subcore_axis_name="s"),
        scratch_types=[
            pltpu.VMEM((N_CHUNKS, 2, 128), jnp.int32),
            pltpu.VMEM((N_CHUNKS, 2, 128), jnp.int32),
            pltpu.VMEM((NBUF, 2, 128, DIM), jnp.float32),
            pltpu.SemaphoreType.DMA((NBUF,)),
            pltpu.SemaphoreType.DMA((NBUF,)),
            pltpu.SemaphoreType.DMA((NBUF,)),
        ],
        compiler_params=pltpu.CompilerParams(use_tc_tiling_on_sc=False),
    )
    out_t = run(xp, pos, table, pe)
    return jnp.swapaxes(out_t, 0, 1)


# speedup vs baseline: 1.2196x; 1.0940x over previous
import math

import jax
import jax.numpy as jnp
import numpy as np
from jax import lax
from jax.experimental import pallas as pl
from jax.experimental.pallas import tpu as pltpu
from jax.experimental.pallas import tpu_sc as plsc

VOCAB_SIZE = 1000000
DIM = 64
SEQ_LEN = 200
BATCH = 1024

NUM_WORKERS = 32
TOTAL_ROWS = BATCH * SEQ_LEN
ROWS_PER_WORKER = TOTAL_ROWS // NUM_WORKERS
CHUNK = 128
N_CHUNKS = ROWS_PER_WORKER // CHUNK
NBUF = 10
D_PE = 3
D_WR = 6


def _position_embedding_np():
    i = np.arange(SEQ_LEN, dtype=np.float64)[:, None]
    j = np.arange(DIM, dtype=np.float64)[None, :]
    even_mask = (np.arange(DIM) % 2 == 0)[None, :]
    temp_even = np.exp(-(j / DIM) * math.log(10000.0))
    temp_odd = np.exp(-((j - 1.0) / DIM) * math.log(10000.0))
    pe = np.where(even_mask, np.sin(i * temp_even), np.cos(i * temp_odd))
    return pe.astype(np.float32)


_PE = _position_embedding_np()

_POS = ((np.arange(N_CHUNKS * CHUNK) % SEQ_LEN)
        .astype(np.int32).reshape(N_CHUNKS, CHUNK))


def _sc_body(idx_hbm, pos_hbm, table_hbm, pe_hbm, out_hbm,
             idx_v, pos_v, rows_v, gsem, psem, wsem):
    wid = lax.axis_index("s") * 2 + lax.axis_index("c")
    wstart = wid * ROWS_PER_WORKER

    pltpu.sync_copy(idx_hbm.at[wid], idx_v)
    pltpu.sync_copy(pos_hbm, pos_v)

    def slot(g):
        return lax.rem(g, NBUF)

    def fire_tbl(g):
        pltpu.async_copy(table_hbm.at[idx_v.at[g]], rows_v.at[slot(g)],
                         gsem.at[slot(g)])

    def wait_tbl(g):
        pltpu.make_async_copy(table_hbm.at[idx_v.at[g]], rows_v.at[slot(g)],
                              gsem.at[slot(g)]).wait()

    def fire_pe(g):
        pltpu.async_copy(pe_hbm.at[pos_v.at[g]], rows_v.at[slot(g)],
                         psem.at[slot(g)], add=True)

    def wait_pe(g):
        pltpu.make_async_copy(pe_hbm.at[pos_v.at[g]], rows_v.at[slot(g)],
                              psem.at[slot(g)]).wait()

    def fire_write(g):
        pltpu.async_copy(rows_v.at[slot(g)],
                         out_hbm.at[pl.ds(wstart + g * CHUNK, CHUNK)],
                         wsem.at[slot(g)])

    def wait_write(g):
        pltpu.make_async_copy(rows_v.at[slot(g)],
                              out_hbm.at[pl.ds(wstart + g * CHUNK, CHUNK)],
                              wsem.at[slot(g)]).wait()

    for i in range(NBUF):
        fire_tbl(i)
        if i >= D_PE:
            wait_tbl(i - D_PE)
            fire_pe(i - D_PE)
        if i >= D_WR:
            wait_pe(i - D_WR)
            fire_write(i - D_WR)

    def steady(i, carry):
        wait_write(i - NBUF)
        fire_tbl(i)
        wait_tbl(i - D_PE)
        fire_pe(i - D_PE)
        wait_pe(i - D_WR)
        fire_write(i - D_WR)
        return carry

    lax.fori_loop(NBUF, N_CHUNKS, steady, 0)

    for g in range(N_CHUNKS - D_PE, N_CHUNKS):
        wait_tbl(g)
        fire_pe(g)
    for g in range(N_CHUNKS - D_WR, N_CHUNKS):
        wait_pe(g)
        fire_write(g)
    for g in range(N_CHUNKS - NBUF, N_CHUNKS):
        wait_write(g)


@jax.jit
def kernel(x, table):
    idx = x.reshape(NUM_WORKERS, N_CHUNKS, CHUNK)
    pe = jnp.asarray(_PE)
    pos = jnp.asarray(_POS)
    run = pl.kernel(
        _sc_body,
        out_type=jax.ShapeDtypeStruct((TOTAL_ROWS, DIM), jnp.float32),
        mesh=plsc.VectorSubcoreMesh(core_axis_name="c", subcore_axis_name="s"),
        scratch_types=[
            pltpu.VMEM((N_CHUNKS, CHUNK), jnp.int32),
            pltpu.VMEM((N_CHUNKS, CHUNK), jnp.int32),
            pltpu.VMEM((NBUF, CHUNK, DIM), jnp.float32),
            pltpu.SemaphoreType.DMA((NBUF,)),
            pltpu.SemaphoreType.DMA((NBUF,)),
            pltpu.SemaphoreType.DMA((NBUF,)),
        ],
        compiler_params=pltpu.CompilerParams(use_tc_tiling_on_sc=False),
    )
    out = run(idx, pos, table, pe)
    return out.reshape(BATCH, SEQ_LEN, DIM)


# speedup vs baseline: 1.5775x; 1.2934x over previous
import math

import jax
import jax.numpy as jnp
import numpy as np
from jax import lax
from jax.experimental import pallas as pl
from jax.experimental.pallas import tpu as pltpu
from jax.experimental.pallas import tpu_sc as plsc

VOCAB_SIZE = 1000000
DIM = 64
SEQ_LEN = 200
BATCH = 1024

NUM_WORKERS = 32
TOTAL_ROWS = BATCH * SEQ_LEN
ROWS_PER_WORKER = TOTAL_ROWS // NUM_WORKERS
CHUNK = 128
N_CHUNKS = ROWS_PER_WORKER // CHUNK
NBUF = 10
D_PE = 3
D_WR = 6


def _position_embedding_np():
    i = np.arange(SEQ_LEN, dtype=np.float64)[:, None]
    j = np.arange(DIM, dtype=np.float64)[None, :]
    even_mask = (np.arange(DIM) % 2 == 0)[None, :]
    temp_even = np.exp(-(j / DIM) * math.log(10000.0))
    temp_odd = np.exp(-((j - 1.0) / DIM) * math.log(10000.0))
    pe = np.where(even_mask, np.sin(i * temp_even), np.cos(i * temp_odd))
    return pe.astype(np.float32)


_PE = _position_embedding_np()

_POS = ((np.arange(N_CHUNKS * CHUNK) % SEQ_LEN)
        .astype(np.int32).reshape(N_CHUNKS, CHUNK))


def _sc_body(idx_hbm, pos_hbm, table_hbm, pe_hbm, out_hbm,
             idx_v, pos_v, rows_v, gsem, psem, wsem):
    wid = lax.axis_index("s") * 2 + lax.axis_index("c")
    wstart = wid * ROWS_PER_WORKER

    pltpu.sync_copy(idx_hbm.at[wid], idx_v)
    pltpu.sync_copy(pos_hbm, pos_v)

    def slot(g):
        return lax.rem(g, NBUF)

    def fire_tbl(g):
        pltpu.async_copy(table_hbm.at[idx_v.at[g]], rows_v.at[slot(g)],
                         gsem.at[slot(g)])

    def wait_tbl(g):
        pltpu.make_async_copy(table_hbm.at[idx_v.at[g]], rows_v.at[slot(g)],
                              gsem.at[slot(g)]).wait()

    def fire_pe(g):
        pltpu.async_copy(pe_hbm.at[pos_v.at[g]], rows_v.at[slot(g)],
                         psem.at[slot(g)], add=True)

    def wait_pe(g):
        pltpu.make_async_copy(pe_hbm.at[pos_v.at[g]], rows_v.at[slot(g)],
                              psem.at[slot(g)]).wait()

    def fire_write(g):
        pltpu.async_copy(rows_v.at[slot(g)],
                         out_hbm.at[pl.ds(wstart + g * CHUNK, CHUNK)],
                         wsem.at[slot(g)])

    def wait_write(g):
        pltpu.make_async_copy(rows_v.at[slot(g)],
                              out_hbm.at[pl.ds(wstart + g * CHUNK, CHUNK)],
                              wsem.at[slot(g)]).wait()

    for i in range(NBUF):
        fire_tbl(i)
        if i >= D_PE:
            wait_tbl(i - D_PE)
            fire_pe(i - D_PE)
        if i >= D_WR:
            wait_pe(i - D_WR)
            fire_write(i - D_WR)

    def steady(i, carry):
        wait_write(i - NBUF)
        fire_tbl(i)
        wait_tbl(i - D_PE)
        fire_pe(i - D_PE)
        wait_pe(i - D_WR)
        fire_write(i - D_WR)
        return carry

    lax.fori_loop(NBUF, N_CHUNKS, steady, 0)

    for g in range(N_CHUNKS - D_PE, N_CHUNKS):
        wait_tbl(g)
        fire_pe(g)
    for g in range(N_CHUNKS - D_WR, N_CHUNKS):
        wait_pe(g)
        fire_write(g)
    for g in range(N_CHUNKS - NBUF, N_CHUNKS):
        wait_write(g)


TCOLS = 8192


def _tc_transpose_body(tt_ref, out_ref):
    blk_t = tt_ref[...].T.reshape(TCOLS // 2, 2, DIM)
    out_ref[:, 0:DIM] = blk_t[:, 0, :]
    out_ref[:, DIM:2 * DIM] = blk_t[:, 1, :]


def _tc_transpose(tt):
    return pl.pallas_call(
        _tc_transpose_body,
        grid=((VOCAB_SIZE + TCOLS - 1) // TCOLS,),
        in_specs=[pl.BlockSpec((DIM, TCOLS), lambda i: (0, i))],
        out_specs=pl.BlockSpec((TCOLS // 2, 2 * DIM), lambda i: (i, 0)),
        out_shape=jax.ShapeDtypeStruct((VOCAB_SIZE // 2, 2 * DIM), jnp.float32),
    )(tt)


@jax.jit
def kernel(x, table):
    table_rows = _tc_transpose(table.T).reshape(VOCAB_SIZE, DIM)
    idx = x.reshape(NUM_WORKERS, N_CHUNKS, CHUNK)
    pe = jnp.asarray(_PE)
    pos = jnp.asarray(_POS)
    run = pl.kernel(
        _sc_body,
        out_type=jax.ShapeDtypeStruct((TOTAL_ROWS, DIM), jnp.float32),
        mesh=plsc.VectorSubcoreMesh(core_axis_name="c", subcore_axis_name="s"),
        scratch_types=[
            pltpu.VMEM((N_CHUNKS, CHUNK), jnp.int32),
            pltpu.VMEM((N_CHUNKS, CHUNK), jnp.int32),
            pltpu.VMEM((NBUF, CHUNK, DIM), jnp.float32),
            pltpu.SemaphoreType.DMA((NBUF,)),
            pltpu.SemaphoreType.DMA((NBUF,)),
            pltpu.SemaphoreType.DMA((NBUF,)),
        ],
        compiler_params=pltpu.CompilerParams(use_tc_tiling_on_sc=False),
    )
    out = run(idx, pos, table_rows, pe)
    return out.reshape(BATCH, SEQ_LEN, DIM)
